# baseline (speedup 1.0000x reference)
import functools

import jax
import jax.numpy as jnp
from jax import lax
from jax.experimental import pallas as pl
from jax.experimental.pallas import tpu as pltpu
from jax.experimental.pallas import tpu_sc as plsc

EMB_DIM = 32
BATCH = 16384
NUM_WORKERS = 32
B_PER_W = BATCH // NUM_WORKERS
N_CHUNKS = 4
CHUNK = B_PER_W // N_CHUNKS
L = 16
N_GROUPS = B_PER_W // L


def _loss_from_score(s):
    z = jnp.exp(-jnp.abs(s))
    w = z / (z + 2.0)
    w2 = w * w
    p = jnp.full_like(w2, 1.0 / 9.0)
    p = 1.0 / 7.0 + w2 * p
    p = 1.0 / 5.0 + w2 * p
    p = 1.0 / 3.0 + w2 * p
    log1p_z = 2.0 * w * (1.0 + w2 * p)
    return jnp.maximum(-s, 0.0) + log1p_z


def _sc_kernel(u_idx_hbm, v_idx_hbm, u_emb_hbm, v_emb_hbm, out_hbm,
               idx_u, idx_v, u_rows, v_rows, out_v, sem):
    cid = lax.axis_index("c")
    sid = lax.axis_index("s")
    wid = sid * 2 + cid

    row0 = wid * N_CHUNKS
    pltpu.sync_copy(u_idx_hbm.at[pl.ds(row0, N_CHUNKS)], idx_u)
    pltpu.sync_copy(v_idx_hbm.at[pl.ds(row0, N_CHUNKS)], idx_v)

    copies = []
    for j in range(N_CHUNKS):
        copies.append(pltpu.async_copy(
            u_emb_hbm.at[idx_u.at[j]], u_rows.at[pl.ds(j * CHUNK, CHUNK)], sem))
        copies.append(pltpu.async_copy(
            v_emb_hbm.at[idx_v.at[j]], v_rows.at[pl.ds(j * CHUNK, CHUNK)], sem))
    for cp in copies:
        cp.wait()

    lane = lax.iota(jnp.int32, L)

    def group_body(g, carry):
        rows = g * L + lane
        acc = jnp.zeros((L,), jnp.float32)
        for d in range(EMB_DIM):
            col = jnp.full((L,), d, jnp.int32)
            ug = plsc.load_gather(u_rows, [rows, col])
            vg = plsc.load_gather(v_rows, [rows, col])
            acc = acc + ug * vg
        off = pl.multiple_of(g * L, L)
        out_v[pl.ds(off, L)] = _loss_from_score(acc)
        return carry

    lax.fori_loop(0, N_GROUPS, group_body, 0)

    base = wid * B_PER_W
    pltpu.sync_copy(out_v, out_hbm.at[pl.ds(base, B_PER_W)])


@jax.jit
def _run(u_idx, v_idx, u_embeddings, v_embeddings):
    mesh = plsc.VectorSubcoreMesh(core_axis_name="c", subcore_axis_name="s")
    kern = functools.partial(
        pl.kernel,
        mesh=mesh,
        compiler_params=pltpu.CompilerParams(
            needs_layout_passes=False, use_tc_tiling_on_sc=False),
        out_type=jax.ShapeDtypeStruct((BATCH,), jnp.float32),
        scratch_types=[
            pltpu.VMEM((N_CHUNKS, CHUNK), jnp.int32),
            pltpu.VMEM((N_CHUNKS, CHUNK), jnp.int32),
            pltpu.VMEM((B_PER_W, EMB_DIM), jnp.float32),
            pltpu.VMEM((B_PER_W, EMB_DIM), jnp.float32),
            pltpu.VMEM((B_PER_W,), jnp.float32),
            pltpu.SemaphoreType.DMA,
        ],
    )(_sc_kernel)
    return kern(u_idx, v_idx, u_embeddings, v_embeddings)


def kernel(edge, negative_edges, u_embeddings, v_embeddings):
    del negative_edges
    u_idx = edge[0].reshape(BATCH // CHUNK, CHUNK)
    v_idx = edge[1].reshape(BATCH // CHUNK, CHUNK)
    return _run(u_idx, v_idx, u_embeddings, v_embeddings)


# speedup vs baseline: 3.2052x; 3.2052x over previous
import functools

import jax
import jax.numpy as jnp
from jax import lax
from jax.experimental import pallas as pl
from jax.experimental.pallas import tpu as pltpu
from jax.experimental.pallas import tpu_sc as plsc

EMB_SIZE = 1000000
EMB_DIM = 32
BATCH = 16384
NUM_WORKERS = 32
B_PER_W = BATCH // NUM_WORKERS
L = 16
W = 128
E_PER_C = 4
N_CHUNKS = B_PER_W // E_PER_C


def _loss_from_score(s):
    z = jnp.exp(-jnp.abs(s))
    w = z / (z + 2.0)
    w2 = w * w
    p = jnp.full_like(w2, 1.0 / 9.0)
    p = 1.0 / 7.0 + w2 * p
    p = 1.0 / 5.0 + w2 * p
    p = 1.0 / 3.0 + w2 * p
    log1p_z = 2.0 * w * (1.0 + w2 * p)
    return jnp.maximum(-s, 0.0) + log1p_z


def _sc_kernel(u_idx_hbm, v_idx_hbm, ut_hbm, vt_hbm, out_hbm,
               idx_u, idx_v, slab_u, slab_v, out_v, sem):
    cid = lax.axis_index("c")
    sid = lax.axis_index("s")
    wid = sid * 2 + cid

    pltpu.sync_copy(u_idx_hbm.at[wid], idx_u)
    pltpu.sync_copy(v_idx_hbm.at[wid], idx_v)

    lane = lax.iota(jnp.int32, L)
    dlo = lax.iota(jnp.int32, L)
    dhi = dlo + L

    def edge_ids(m):
        off = pl.multiple_of((m >> 2) * L, L)
        ru = idx_u[pl.ds(off, L)]
        rv = idx_v[pl.ds(off, L)]
        return ru, rv

    def pick(vec, pos):
        return jnp.sum(jnp.where(lane == pos, vec, 0))

    def fire(m):
        slot0 = (m & 1) * E_PER_C
        ru, rv = edge_ids(m)
        for i in range(E_PER_C):
            e = (m & 3) * E_PER_C + i
            ku = pl.multiple_of((pick(ru, e) >> 7) << 7, W)
            kv = pl.multiple_of((pick(rv, e) >> 7) << 7, W)
            pltpu.async_copy(
                ut_hbm.at[:, pl.ds(ku, W)], slab_u.at[slot0 + i], sem)
            pltpu.async_copy(
                vt_hbm.at[:, pl.ds(kv, W)], slab_v.at[slot0 + i], sem)

    fire(0)

    def chunk_body(n, score):
        slot0 = (n & 1) * E_PER_C
        for i in range(E_PER_C):
            pltpu.make_async_copy(
                ut_hbm.at[:, pl.ds(0, W)], slab_u.at[slot0 + i], sem).wait()
            pltpu.make_async_copy(
                vt_hbm.at[:, pl.ds(0, W)], slab_v.at[slot0 + i], sem).wait()

        @pl.when(n + 1 < N_CHUNKS)
        def _():
            fire(n + 1)

        ru, rv = edge_ids(n)
        for i in range(E_PER_C):
            e = (n & 3) * E_PER_C + i
            cu = jnp.full((L,), pick(ru, e) & 127, jnp.int32)
            cv = jnp.full((L,), pick(rv, e) & 127, jnp.int32)
            svec = jnp.full((L,), slot0 + i, jnp.int32)
            u1 = plsc.load_gather(slab_u, [svec, dlo, cu])
            u2 = plsc.load_gather(slab_u, [svec, dhi, cu])
            v1 = plsc.load_gather(slab_v, [svec, dlo, cv])
            v2 = plsc.load_gather(slab_v, [svec, dhi, cv])
            s = jnp.sum(u1 * v1 + u2 * v2)
            score = jnp.where(lane == (n & 3) * E_PER_C + i, s, score)
        @pl.when((n & 3) == 3)
        def _():
            off = pl.multiple_of((n >> 2) * L, L)
            out_v[pl.ds(off, L)] = _loss_from_score(score)
        return score

    lax.fori_loop(0, N_CHUNKS, chunk_body, jnp.zeros((L,), jnp.float32))

    base = wid * B_PER_W
    pltpu.sync_copy(out_v, out_hbm.at[pl.ds(base, B_PER_W)])


@jax.jit
def _run(u_idx, v_idx, ut, vt):
    mesh = plsc.VectorSubcoreMesh(core_axis_name="c", subcore_axis_name="s")
    kern = functools.partial(
        pl.kernel,
        mesh=mesh,
        compiler_params=pltpu.CompilerParams(
            needs_layout_passes=False, use_tc_tiling_on_sc=True),
        out_type=jax.ShapeDtypeStruct((BATCH,), jnp.float32),
        scratch_types=[
            pltpu.VMEM((B_PER_W,), jnp.int32),
            pltpu.VMEM((B_PER_W,), jnp.int32),
            pltpu.VMEM((2 * E_PER_C, EMB_DIM, W), jnp.float32),
            pltpu.VMEM((2 * E_PER_C, EMB_DIM, W), jnp.float32),
            pltpu.VMEM((B_PER_W,), jnp.float32),
            pltpu.SemaphoreType.DMA,
        ],
    )(_sc_kernel)
    return kern(u_idx, v_idx, ut, vt)


def kernel(edge, negative_edges, u_embeddings, v_embeddings):
    del negative_edges
    u_idx = edge[0].reshape(NUM_WORKERS, B_PER_W)
    v_idx = edge[1].reshape(NUM_WORKERS, B_PER_W)
    return _run(u_idx, v_idx, u_embeddings.T, v_embeddings.T)


# speedup vs baseline: 3.7005x; 1.1545x over previous
import functools

import jax
import jax.numpy as jnp
from jax import lax
from jax.experimental import pallas as pl
from jax.experimental.pallas import tpu as pltpu
from jax.experimental.pallas import tpu_sc as plsc

EMB_SIZE = 1000000
EMB_DIM = 32
BATCH = 16384
NUM_WORKERS = 32
B_PER_W = BATCH // NUM_WORKERS
L = 16
W = 128
E_PER_C = 4
N_CHUNKS = B_PER_W // E_PER_C


def _loss_from_score(s):
    z = jnp.exp(-jnp.abs(s))
    w = z / (z + 2.0)
    w2 = w * w
    p = jnp.full_like(w2, 1.0 / 9.0)
    p = 1.0 / 7.0 + w2 * p
    p = 1.0 / 5.0 + w2 * p
    p = 1.0 / 3.0 + w2 * p
    log1p_z = 2.0 * w * (1.0 + w2 * p)
    return jnp.maximum(-s, 0.0) + log1p_z


def _sc_kernel(u_idx_hbm, v_idx_hbm, ut_hbm, vt_hbm, out_hbm,
               idx_u, idx_v, slab_u, slab_v, out_v, sem_a, sem_b):
    cid = lax.axis_index("c")
    sid = lax.axis_index("s")
    wid = sid * 2 + cid

    pltpu.sync_copy(u_idx_hbm.at[wid], idx_u)
    pltpu.sync_copy(v_idx_hbm.at[wid], idx_v)

    lane = lax.iota(jnp.int32, L)
    dlo = lax.iota(jnp.int32, L)
    dhi = dlo + L

    def edge_ids(m):
        off = pl.multiple_of((m >> 2) * L, L)
        ru = idx_u[pl.ds(off, L)]
        rv = idx_v[pl.ds(off, L)]
        return ru, rv

    def pick(vec, pos):
        return jnp.sum(jnp.where(lane == pos, vec, 0))

    def fire(m, sem):
        slot0 = (m & 1) * E_PER_C
        ru, rv = edge_ids(m)
        for i in range(E_PER_C):
            e = (m & 3) * E_PER_C + i
            ku = pl.multiple_of((pick(ru, e) >> 7) << 7, W)
            kv = pl.multiple_of((pick(rv, e) >> 7) << 7, W)
            pltpu.async_copy(
                ut_hbm.at[:, pl.ds(ku, W)], slab_u.at[slot0 + i], sem)
            pltpu.async_copy(
                vt_hbm.at[:, pl.ds(kv, W)], slab_v.at[slot0 + i], sem)

    def drain(n, sem):
        slot0 = (n & 1) * E_PER_C
        for i in range(E_PER_C):
            pltpu.make_async_copy(
                ut_hbm.at[:, pl.ds(0, W)], slab_u.at[slot0 + i], sem).wait()
            pltpu.make_async_copy(
                ut_hbm.at[:, pl.ds(0, W)], slab_v.at[slot0 + i], sem).wait()

    def compute(n, score):
        slot0 = (n & 1) * E_PER_C
        ru, rv = edge_ids(n)
        for i in range(E_PER_C):
            e = (n & 3) * E_PER_C + i
            cu = jnp.full((L,), pick(ru, e) & 127, jnp.int32)
            cv = jnp.full((L,), pick(rv, e) & 127, jnp.int32)
            svec = jnp.full((L,), slot0 + i, jnp.int32)
            u1 = plsc.load_gather(slab_u, [svec, dlo, cu])
            u2 = plsc.load_gather(slab_u, [svec, dhi, cu])
            v1 = plsc.load_gather(slab_v, [svec, dlo, cv])
            v2 = plsc.load_gather(slab_v, [svec, dhi, cv])
            s = jnp.sum(u1 * v1 + u2 * v2)
            score = jnp.where(lane == (n & 3) * E_PER_C + i, s, score)
        @pl.when((n & 3) == 3)
        def _():
            off = pl.multiple_of((n >> 2) * L, L)
            out_v[pl.ds(off, L)] = _loss_from_score(score)
        return score

    fire(0, sem_a)

    def pair_body(m, score):
        p = 2 * m
        q = p + 1
        fire(q, sem_b)
        drain(p, sem_a)
        score = compute(p, score)

        @pl.when(q + 1 < N_CHUNKS)
        def _():
            fire(q + 1, sem_a)

        drain(q, sem_b)
        score = compute(q, score)
        return score

    lax.fori_loop(0, N_CHUNKS // 2, pair_body, jnp.zeros((L,), jnp.float32))

    base = wid * B_PER_W
    pltpu.sync_copy(out_v, out_hbm.at[pl.ds(base, B_PER_W)])


@jax.jit
def _run(u_idx, v_idx, ut, vt):
    mesh = plsc.VectorSubcoreMesh(core_axis_name="c", subcore_axis_name="s")
    kern = functools.partial(
        pl.kernel,
        mesh=mesh,
        compiler_params=pltpu.CompilerParams(
            needs_layout_passes=False, use_tc_tiling_on_sc=True),
        out_type=jax.ShapeDtypeStruct((BATCH,), jnp.float32),
        scratch_types=[
            pltpu.VMEM((B_PER_W,), jnp.int32),
            pltpu.VMEM((B_PER_W,), jnp.int32),
            pltpu.VMEM((2 * E_PER_C, EMB_DIM, W), jnp.float32),
            pltpu.VMEM((2 * E_PER_C, EMB_DIM, W), jnp.float32),
            pltpu.VMEM((B_PER_W,), jnp.float32),
            pltpu.SemaphoreType.DMA,
            pltpu.SemaphoreType.DMA,
        ],
    )(_sc_kernel)
    return kern(u_idx, v_idx, ut, vt)


def kernel(edge, negative_edges, u_embeddings, v_embeddings):
    del negative_edges
    u_idx = edge[0].reshape(NUM_WORKERS, B_PER_W)
    v_idx = edge[1].reshape(NUM_WORKERS, B_PER_W)
    return _run(u_idx, v_idx, u_embeddings.T, v_embeddings.T)


# speedup vs baseline: 3.9659x; 1.0717x over previous
import functools

import jax
import jax.numpy as jnp
from jax import lax
from jax.experimental import pallas as pl
from jax.experimental.pallas import tpu as pltpu
from jax.experimental.pallas import tpu_sc as plsc

EMB_SIZE = 1000000
EMB_DIM = 32
BATCH = 16384
NUM_WORKERS = 32
B_PER_W = BATCH // NUM_WORKERS
L = 16
W = 128
E_PER_C = 4
N_CHUNKS = B_PER_W // E_PER_C


def _loss_from_score(s):
    z = jnp.exp(-jnp.abs(s))
    w = z / (z + 2.0)
    w2 = w * w
    p = jnp.full_like(w2, 1.0 / 9.0)
    p = 1.0 / 7.0 + w2 * p
    p = 1.0 / 5.0 + w2 * p
    p = 1.0 / 3.0 + w2 * p
    log1p_z = 2.0 * w * (1.0 + w2 * p)
    return jnp.maximum(-s, 0.0) + log1p_z


def _sc_kernel(u_idx_hbm, v_idx_hbm, ut_hbm, vt_hbm, out_hbm,
               idx_u, idx_v, slab_u, slab_v, out_v, sem_a, sem_b, sem_c):
    cid = lax.axis_index("c")
    sid = lax.axis_index("s")
    wid = sid * 2 + cid

    pltpu.sync_copy(u_idx_hbm.at[wid], idx_u)
    pltpu.sync_copy(v_idx_hbm.at[wid], idx_v)

    lane = lax.iota(jnp.int32, L)
    dlo = lax.iota(jnp.int32, L)
    dhi = dlo + L

    def edge_ids(m):
        off = pl.multiple_of((m >> 2) * L, L)
        ru = idx_u[pl.ds(off, L)]
        rv = idx_v[pl.ds(off, L)]
        return ru, rv

    def pick(vec, pos):
        return jnp.sum(jnp.where(lane == pos, vec, 0))

    def fire(m, b, sem):
        slot0 = b * E_PER_C
        ru, rv = edge_ids(m)
        for i in range(E_PER_C):
            e = (m & 3) * E_PER_C + i
            ku = pl.multiple_of((pick(ru, e) >> 7) << 7, W)
            kv = pl.multiple_of((pick(rv, e) >> 7) << 7, W)
            pltpu.async_copy(
                ut_hbm.at[:, pl.ds(ku, W)], slab_u.at[slot0 + i], sem)
            pltpu.async_copy(
                vt_hbm.at[:, pl.ds(kv, W)], slab_v.at[slot0 + i], sem)

    def drain(b, sem):
        slot0 = b * E_PER_C
        for i in range(E_PER_C):
            pltpu.make_async_copy(
                ut_hbm.at[:, pl.ds(0, W)], slab_u.at[slot0 + i], sem).wait()
            pltpu.make_async_copy(
                ut_hbm.at[:, pl.ds(0, W)], slab_v.at[slot0 + i], sem).wait()

    def compute(n, b, score):
        slot0 = b * E_PER_C
        ru, rv = edge_ids(n)
        for i in range(E_PER_C):
            e = (n & 3) * E_PER_C + i
            cu = jnp.full((L,), pick(ru, e) & 127, jnp.int32)
            cv = jnp.full((L,), pick(rv, e) & 127, jnp.int32)
            svec = jnp.full((L,), slot0 + i, jnp.int32)
            u1 = plsc.load_gather(slab_u, [svec, dlo, cu])
            u2 = plsc.load_gather(slab_u, [svec, dhi, cu])
            v1 = plsc.load_gather(slab_v, [svec, dlo, cv])
            v2 = plsc.load_gather(slab_v, [svec, dhi, cv])
            s = jnp.sum(u1 * v1 + u2 * v2)
            score = jnp.where(lane == (n & 3) * E_PER_C + i, s, score)
        @pl.when((n & 3) == 3)
        def _():
            off = pl.multiple_of((n >> 2) * L, L)
            out_v[pl.ds(off, L)] = _loss_from_score(score)
        return score

    fire(0, 0, sem_a)
    fire(1, 1, sem_b)

    def tri_body(t, score):
        c0 = 3 * t
        fire(c0 + 2, 2, sem_c)
        drain(0, sem_a)
        score = compute(c0, 0, score)

        @pl.when(c0 + 3 < N_CHUNKS)
        def _():
            fire(c0 + 3, 0, sem_a)

        drain(1, sem_b)
        score = compute(c0 + 1, 1, score)

        @pl.when(c0 + 4 < N_CHUNKS)
        def _():
            fire(c0 + 4, 1, sem_b)

        drain(2, sem_c)
        score = compute(c0 + 2, 2, score)
        return score

    n_tri = N_CHUNKS // 3
    score = lax.fori_loop(0, n_tri, tri_body,
                          jnp.zeros((L,), jnp.float32))
    drain(0, sem_a)
    score = compute(N_CHUNKS - 2, 0, score)
    drain(1, sem_b)
    compute(N_CHUNKS - 1, 1, score)

    base = wid * B_PER_W
    pltpu.sync_copy(out_v, out_hbm.at[pl.ds(base, B_PER_W)])


@jax.jit
def _run(u_idx, v_idx, ut, vt):
    mesh = plsc.VectorSubcoreMesh(core_axis_name="c", subcore_axis_name="s")
    kern = functools.partial(
        pl.kernel,
        mesh=mesh,
        compiler_params=pltpu.CompilerParams(
            needs_layout_passes=False, use_tc_tiling_on_sc=True),
        out_type=jax.ShapeDtypeStruct((BATCH,), jnp.float32),
        scratch_types=[
            pltpu.VMEM((B_PER_W,), jnp.int32),
            pltpu.VMEM((B_PER_W,), jnp.int32),
            pltpu.VMEM((3 * E_PER_C, EMB_DIM, W), jnp.float32),
            pltpu.VMEM((3 * E_PER_C, EMB_DIM, W), jnp.float32),
            pltpu.VMEM((B_PER_W,), jnp.float32),
            pltpu.SemaphoreType.DMA,
            pltpu.SemaphoreType.DMA,
            pltpu.SemaphoreType.DMA,
        ],
    )(_sc_kernel)
    return kern(u_idx, v_idx, ut, vt)


def kernel(edge, negative_edges, u_embeddings, v_embeddings):
    del negative_edges
    u_idx = edge[0].reshape(NUM_WORKERS, B_PER_W)
    v_idx = edge[1].reshape(NUM_WORKERS, B_PER_W)
    return _run(u_idx, v_idx, u_embeddings.T, v_embeddings.T)


# speedup vs baseline: 4.0180x; 1.0131x over previous
import functools

import jax
import jax.numpy as jnp
from jax import lax
from jax.experimental import pallas as pl
from jax.experimental.pallas import tpu as pltpu
from jax.experimental.pallas import tpu_sc as plsc

EMB_SIZE = 1000000
EMB_DIM = 32
BATCH = 16384
NUM_WORKERS = 32
B_PER_W = BATCH // NUM_WORKERS
L = 16
W = 128
E_PER_C = 4
N_CHUNKS = B_PER_W // E_PER_C


def _loss_from_score(s):
    z = jnp.exp(-jnp.abs(s))
    w = z / (z + 2.0)
    w2 = w * w
    p = jnp.full_like(w2, 1.0 / 9.0)
    p = 1.0 / 7.0 + w2 * p
    p = 1.0 / 5.0 + w2 * p
    p = 1.0 / 3.0 + w2 * p
    log1p_z = 2.0 * w * (1.0 + w2 * p)
    return jnp.maximum(-s, 0.0) + log1p_z


def _sc_kernel(u_idx_hbm, v_idx_hbm, ut_hbm, vt_hbm, out_hbm,
               idx_u, idx_v, slab_u, slab_v, out_v, sem_a, sem_b, sem_c):
    cid = lax.axis_index("c")
    sid = lax.axis_index("s")
    wid = sid * 2 + cid

    pltpu.sync_copy(u_idx_hbm.at[wid], idx_u)
    pltpu.sync_copy(v_idx_hbm.at[wid], idx_v)

    lane = lax.iota(jnp.int32, L)
    dlo = lax.iota(jnp.int32, L)
    dhi = dlo + L

    def edge_ids(m):
        off = pl.multiple_of((m >> 2) * L, L)
        ru = idx_u[pl.ds(off, L)]
        rv = idx_v[pl.ds(off, L)]
        return ru, rv

    def pick(vec, pos):
        return jnp.sum(jnp.where(lane == pos, vec, 0))

    def fire(m, b, sem):
        slot0 = b * E_PER_C
        ru, rv = edge_ids(m)
        for i in range(E_PER_C):
            e = (m & 3) * E_PER_C + i
            ku = pl.multiple_of((pick(ru, e) >> 7) << 7, W)
            kv = pl.multiple_of((pick(rv, e) >> 7) << 7, W)
            for blk in range(EMB_DIM // 8):
                pltpu.async_copy(
                    ut_hbm.at[pl.ds(8 * blk, 8), pl.ds(ku, W)],
                    slab_u.at[slot0 + i, blk], sem)
                pltpu.async_copy(
                    vt_hbm.at[pl.ds(8 * blk, 8), pl.ds(kv, W)],
                    slab_v.at[slot0 + i, blk], sem)

    def drain(b, sem):
        slot0 = b * E_PER_C
        for i in range(E_PER_C):
            pltpu.make_async_copy(
                ut_hbm.at[:, pl.ds(0, W)],
                slab_u.at[slot0 + i].reshape(EMB_DIM, W), sem).wait()
            pltpu.make_async_copy(
                ut_hbm.at[:, pl.ds(0, W)],
                slab_v.at[slot0 + i].reshape(EMB_DIM, W), sem).wait()

    def compute(n, b, score):
        slot0 = b * E_PER_C
        ru, rv = edge_ids(n)
        for i in range(E_PER_C):
            e = (n & 3) * E_PER_C + i
            cu = jnp.full((L,), pick(ru, e) & 127, jnp.int32)
            cv = jnp.full((L,), pick(rv, e) & 127, jnp.int32)
            svec = jnp.full((L,), slot0 + i, jnp.int32)
            blo = dlo >> 3
            bhi = dhi >> 3
            jlo = dlo & 7
            jhi = dhi & 7
            u1 = plsc.load_gather(slab_u, [svec, blo, jlo, cu])
            u2 = plsc.load_gather(slab_u, [svec, bhi, jhi, cu])
            v1 = plsc.load_gather(slab_v, [svec, blo, jlo, cv])
            v2 = plsc.load_gather(slab_v, [svec, bhi, jhi, cv])
            s = jnp.sum(u1 * v1 + u2 * v2)
            score = jnp.where(lane == (n & 3) * E_PER_C + i, s, score)
        @pl.when((n & 3) == 3)
        def _():
            off = pl.multiple_of((n >> 2) * L, L)
            out_v[pl.ds(off, L)] = _loss_from_score(score)
        return score

    fire(0, 0, sem_a)
    fire(1, 1, sem_b)

    def tri_body(t, score):
        c0 = 3 * t
        fire(c0 + 2, 2, sem_c)
        drain(0, sem_a)
        score = compute(c0, 0, score)

        @pl.when(c0 + 3 < N_CHUNKS)
        def _():
            fire(c0 + 3, 0, sem_a)

        drain(1, sem_b)
        score = compute(c0 + 1, 1, score)

        @pl.when(c0 + 4 < N_CHUNKS)
        def _():
            fire(c0 + 4, 1, sem_b)

        drain(2, sem_c)
        score = compute(c0 + 2, 2, score)
        return score

    n_tri = N_CHUNKS // 3
    score = lax.fori_loop(0, n_tri, tri_body,
                          jnp.zeros((L,), jnp.float32))
    drain(0, sem_a)
    score = compute(N_CHUNKS - 2, 0, score)
    drain(1, sem_b)
    compute(N_CHUNKS - 1, 1, score)

    base = wid * B_PER_W
    pltpu.sync_copy(out_v, out_hbm.at[pl.ds(base, B_PER_W)])


@jax.jit
def _run(u_idx, v_idx, ut, vt):
    mesh = plsc.VectorSubcoreMesh(core_axis_name="c", subcore_axis_name="s")
    kern = functools.partial(
        pl.kernel,
        mesh=mesh,
        compiler_params=pltpu.CompilerParams(
            needs_layout_passes=False, use_tc_tiling_on_sc=True),
        out_type=jax.ShapeDtypeStruct((BATCH,), jnp.float32),
        scratch_types=[
            pltpu.VMEM((B_PER_W,), jnp.int32),
            pltpu.VMEM((B_PER_W,), jnp.int32),
            pltpu.VMEM((3 * E_PER_C, EMB_DIM // 8, 8, W), jnp.float32),
            pltpu.VMEM((3 * E_PER_C, EMB_DIM // 8, 8, W), jnp.float32),
            pltpu.VMEM((B_PER_W,), jnp.float32),
            pltpu.SemaphoreType.DMA,
            pltpu.SemaphoreType.DMA,
            pltpu.SemaphoreType.DMA,
        ],
    )(_sc_kernel)
    return kern(u_idx, v_idx, ut, vt)


def kernel(edge, negative_edges, u_embeddings, v_embeddings):
    del negative_edges
    u_idx = edge[0].reshape(NUM_WORKERS, B_PER_W)
    v_idx = edge[1].reshape(NUM_WORKERS, B_PER_W)
    return _run(u_idx, v_idx, u_embeddings.T, v_embeddings.T)
